# baseline (speedup 1.0000x reference)
import functools

import jax
import jax.numpy as jnp
from jax.experimental import pallas as pl

N = 50000
E = 800000
L = 2
D = 512
K = 64
H = 64
C = 40
B = 512
T = 0.1
NUM_HOPS = 2

ROWS = 2000


def _proj_body(x_ref, w1_ref, b1_ref, g_ref, bln_ref, w2_ref, b2_ref, o_ref):
    x = x_ref[...]
    h = jnp.dot(x, w1_ref[...], preferred_element_type=jnp.float32) + b1_ref[...]
    mu = jnp.mean(h, axis=-1, keepdims=True)
    var = jnp.mean((h - mu) ** 2, axis=-1, keepdims=True)
    h = (h - mu) * jax.lax.rsqrt(var + 1e-5) * g_ref[...] + bln_ref[...]
    h = jnp.maximum(h, 0.0)
    o_ref[...] = jnp.dot(h, w2_ref[...], preferred_element_type=jnp.float32) + b2_ref[...]


@functools.partial(jax.jit, static_argnames=())
def _proj_pallas(x, w1, b1, g, bln, w2, b2):
    n = x.shape[0]
    grid = (n // ROWS,)
    return pl.pallas_call(
        _proj_body,
        grid=grid,
        in_specs=[
            pl.BlockSpec((ROWS, D), lambda i: (i, 0)),
            pl.BlockSpec((D, K), lambda i: (0, 0)),
            pl.BlockSpec((K,), lambda i: (0,)),
            pl.BlockSpec((K,), lambda i: (0,)),
            pl.BlockSpec((K,), lambda i: (0,)),
            pl.BlockSpec((K, K), lambda i: (0, 0)),
            pl.BlockSpec((K,), lambda i: (0,)),
        ],
        out_specs=pl.BlockSpec((ROWS, K), lambda i: (i, 0)),
        out_shape=jax.ShapeDtypeStruct((n, K), jnp.float32),
    )(x, w1, b1, g, bln, w2, b2)


def _k_hop_mask(node_idx, edge_index, num_nodes):
    row = edge_index[1]
    col = edge_index[0]
    cur = jnp.zeros((num_nodes,), dtype=bool).at[node_idx].set(True)
    total = cur
    for _ in range(NUM_HOPS):
        edge_mask = cur[row]
        cur = jnp.zeros((num_nodes,), dtype=bool).at[col].max(edge_mask)
        total = total | cur
    return total


def _gcn_conv(x, src, dst, W, b, n, emask_f, nmask_f, dinv):
    norm = dinv[src] * dinv[dst] * emask_f
    h = x @ W
    out = jnp.zeros((n, W.shape[1]), x.dtype).at[dst].add(h[src] * norm[:, None])
    out = out + h * (dinv * dinv * nmask_f)[:, None]
    return out + b


def kernel(hidden_embeds, node_id, edge_index, params):
    n = hidden_embeds.shape[1]
    nmask = _k_hop_mask(node_id, edge_index, n)
    src, dst = edge_index[0], edge_index[1]
    emask = nmask[src] & nmask[dst]
    emask_f = emask.astype(jnp.float32)
    nmask_f = nmask.astype(jnp.float32)

    deg = jnp.zeros((n,), jnp.float32).at[dst].add(emask_f) + nmask_f
    dinv = jax.lax.rsqrt(jnp.maximum(deg, 1.0))

    last = None
    for i in range(L):
        x = _proj_pallas(
            hidden_embeds[i],
            params[f'proj_W1_{i}'], params[f'proj_b1_{i}'],
            params[f'ln_g_{i}'], params[f'ln_b_{i}'],
            params[f'proj_W2_{i}'], params[f'proj_b2_{i}'],
        )
        if i > 0:
            a = jax.nn.sigmoid(params[f'alpha_{i}'] / T)
            x = x * a + last * (1.0 - a)
        h = jax.nn.relu(_gcn_conv(x, src, dst, params[f'gcn_W1_{i}'],
                                  params[f'gcn_b1_{i}'], n, emask_f, nmask_f, dinv))
        last = _gcn_conv(h, src, dst, params[f'gcn_W2_{i}'],
                         params[f'gcn_b2_{i}'], n, emask_f, nmask_f, dinv)

    nsum = (jnp.zeros((n, last.shape[1]), last.dtype)
            .at[src].add(last[dst] * emask_f[:, None])
            .at[dst].add(last[src] * emask_f[:, None]))
    cnt = jnp.zeros((n,), last.dtype).at[src].add(emask_f).at[dst].add(emask_f)
    pooled = (last + nsum) / (1.0 + cnt)[:, None]
    feat = jnp.concatenate([last[node_id], pooled[node_id]], axis=1)
    return feat @ params['cls_W'] + params['cls_b']


# speedup vs baseline: 1.4807x; 1.4807x over previous
import functools

import jax
import jax.numpy as jnp
from jax import lax
from jax.experimental import pallas as pl
from jax.experimental.pallas import tpu as pltpu
from jax.experimental.pallas import tpu_sc as plsc

N = 50000
E = 800000
L = 2
D = 512
K = 64
C = 40
B = 512
T = 0.1
NUM_HOPS = 2

NQ = 12544
NQP = 12560
TRASH = 12544
EC = 128
NCH = 392
NPP = 200
EPT = EC * NCH
EPAD = EPT * 16
STR = NQP // 16

ROWS = 2000


def _proj_body(x_ref, w1_ref, b1_ref, g_ref, bln_ref, w2_ref, b2_ref, o_ref):
    x = x_ref[...]
    h = jnp.dot(x, w1_ref[...], preferred_element_type=jnp.float32) + b1_ref[...]
    mu = jnp.mean(h, axis=-1, keepdims=True)
    var = jnp.mean((h - mu) ** 2, axis=-1, keepdims=True)
    h = (h - mu) * jax.lax.rsqrt(var + 1e-5) * g_ref[...] + bln_ref[...]
    h = jnp.maximum(h, 0.0)
    o_ref[...] = jnp.dot(h, w2_ref[...], preferred_element_type=jnp.float32) + b2_ref[...]


def _proj_pallas(x, w1, b1, g, bln, w2, b2):
    n = x.shape[0]
    return pl.pallas_call(
        _proj_body,
        grid=(n // ROWS,),
        in_specs=[
            pl.BlockSpec((ROWS, D), lambda i: (i, 0)),
            pl.BlockSpec((D, K), lambda i: (0, 0)),
            pl.BlockSpec((K,), lambda i: (0,)),
            pl.BlockSpec((K,), lambda i: (0,)),
            pl.BlockSpec((K,), lambda i: (0,)),
            pl.BlockSpec((K, K), lambda i: (0, 0)),
            pl.BlockSpec((K,), lambda i: (0,)),
        ],
        out_specs=pl.BlockSpec((ROWS, K), lambda i: (i, 0)),
        out_shape=jax.ShapeDtypeStruct((n, K), jnp.float32),
    )(x, w1, b1, g, bln, w2, b2)


STG = 157


def _spmm_body(m_hbm, pk_hbm, zero_hbm, out_hbm,
               ib0, ib1, rows0, rows1, stage, acc_sh, semi0, semi1, semg0, semg1):
    c = lax.axis_index("c")
    s = lax.axis_index("s")
    pltpu.sync_copy(zero_hbm.at[pl.ds(0, STG)], stage)
    for t in range(5):
        pltpu.sync_copy(stage, acc_sh.at[pl.ds(s * STR + t * STG, STG)])
    pltpu.async_copy(pk_hbm.at[c, s, 0], ib0, semi0)
    plsc.subcore_barrier()

    def one_pair(ib_cur, ib_nxt, semi_cur, semi_nxt, pair, nxt_pair):
        pltpu.make_async_copy(pk_hbm.at[c, s, 0], ib_cur, semi_cur).wait()
        g0 = pltpu.async_copy(m_hbm.at[ib_cur.at[0, 0]], rows0, semg0)
        g1 = pltpu.async_copy(m_hbm.at[ib_cur.at[1, 0]], rows1, semg1)
        pltpu.async_copy(pk_hbm.at[c, s, nxt_pair], ib_nxt, semi_nxt)
        g0.wait()
        pltpu.sync_copy(rows0, acc_sh.at[ib_cur.at[0, 1]], add=True)
        g1.wait()
        pltpu.sync_copy(rows1, acc_sh.at[ib_cur.at[1, 1]], add=True)

    def body(k, carry):
        one_pair(ib0, ib1, semi0, semi1, 2 * k, 2 * k + 1)
        one_pair(ib1, ib0, semi1, semi0, 2 * k + 1, 2 * k + 2)
        return carry

    lax.fori_loop(0, NCH // 4, body, 0)
    pltpu.make_async_copy(pk_hbm.at[c, s, 0], ib0, semi0).wait()
    plsc.subcore_barrier()
    for t in range(5):
        pltpu.sync_copy(acc_sh.at[pl.ds(s * STR + t * STG, STG)], stage)
        pltpu.sync_copy(stage, out_hbm.at[c, pl.ds(s * STR + t * STG, STG)])


def _spmm_pass(m, pk_p, zeros):
    mesh = plsc.VectorSubcoreMesh(core_axis_name="c", subcore_axis_name="s")
    f = pl.kernel(
        _spmm_body,
        mesh=mesh,
        compiler_params=pltpu.CompilerParams(use_tc_tiling_on_sc=False),
        out_type=jax.ShapeDtypeStruct((2, NQP, K), jnp.float32),
        scratch_types=[
            pltpu.VMEM((2, 2, EC), jnp.int32),
            pltpu.VMEM((2, 2, EC), jnp.int32),
            pltpu.VMEM((EC, K), jnp.float32),
            pltpu.VMEM((EC, K), jnp.float32),
            pltpu.VMEM((STG, K), jnp.float32),
            pltpu.VMEM_SHARED((NQP, K), jnp.float32),
            pltpu.SemaphoreType.DMA,
            pltpu.SemaphoreType.DMA,
            pltpu.SemaphoreType.DMA,
            pltpu.SemaphoreType.DMA,
        ],
    )
    return f(m, pk_p, zeros)


def _spmm(m, pk, zeros):
    o0 = _spmm_pass(m, pk[0], zeros)
    o1 = _spmm_pass(m, pk[1], zeros)
    full = jnp.concatenate(
        [o0[0, :NQ], o0[1, :NQ], o1[0, :NQ], o1[1, :NQ]], axis=0)
    return full[:N]


def _pack_idx(gidx, route, valid):
    pad_ch = 2 * NPP - NCH
    g = jnp.zeros((EPAD,), jnp.int32).at[:E].set(gidx).reshape(16, NCH, EC)
    g = jnp.pad(g, ((0, 0), (0, pad_ch), (0, 0)))
    outs = []
    for q in range(4):
        local = route - q * NQ
        ok = valid & (local >= 0) & (local < NQ)
        outs.append(jnp.where(ok, local, TRASH).astype(jnp.int32))
    r = jnp.stack(outs)
    r = jnp.full((4, EPAD), TRASH, jnp.int32).at[:, :E].set(r)
    r = r.reshape(4, 16, NCH, EC)
    r = jnp.pad(r, ((0, 0), (0, 0), (0, pad_ch), (0, 0)),
                constant_values=TRASH)
    r = r.reshape(2, 2, 16, 2 * NPP, EC)
    g4 = jnp.broadcast_to(g[None, None], (2, 2, 16, 2 * NPP, EC))
    pk = jnp.stack([g4, r], axis=4)
    return pk.reshape(2, 2, 16, NPP, 2, 2, EC)


def _k_hop_mask(node_idx, edge_index, num_nodes):
    row = edge_index[1]
    col = edge_index[0]
    cur = jnp.zeros((num_nodes,), dtype=bool).at[node_idx].set(True)
    total = cur
    for _ in range(NUM_HOPS):
        edge_mask = cur[row]
        cur = jnp.zeros((num_nodes,), dtype=bool).at[col].max(edge_mask)
        total = total | cur
    return total


def kernel(hidden_embeds, node_id, edge_index, params):
    n = hidden_embeds.shape[1]
    nmask = _k_hop_mask(node_id, edge_index, n)
    src, dst = edge_index[0], edge_index[1]
    emask = nmask[src] & nmask[dst]
    emask_f = emask.astype(jnp.float32)
    nmask_f = nmask.astype(jnp.float32)

    deg = jnp.zeros((n,), jnp.float32).at[dst].add(emask_f) + nmask_f
    dinv = jax.lax.rsqrt(jnp.maximum(deg, 1.0))

    zeros = jnp.zeros((STG, K), jnp.float32)
    pk_dst = _pack_idx(src, dst, emask)
    pk_src = _pack_idx(dst, src, emask)

    def conv(x, W, b):
        g = (x @ W) * dinv[:, None]
        sfull = _spmm(g, pk_dst, zeros)
        return dinv[:, None] * (sfull + g * nmask_f[:, None]) + b

    last = None
    for i in range(L):
        x = _proj_pallas(
            hidden_embeds[i],
            params[f'proj_W1_{i}'], params[f'proj_b1_{i}'],
            params[f'ln_g_{i}'], params[f'ln_b_{i}'],
            params[f'proj_W2_{i}'], params[f'proj_b2_{i}'],
        )
        if i > 0:
            a = jax.nn.sigmoid(params[f'alpha_{i}'] / T)
            x = x * a + last * (1.0 - a)
        h = jax.nn.relu(conv(x, params[f'gcn_W1_{i}'], params[f'gcn_b1_{i}']))
        last = conv(h, params[f'gcn_W2_{i}'], params[f'gcn_b2_{i}'])

    nsum = (_spmm(last, pk_src, zeros)
            + _spmm(last, pk_dst, zeros))
    cnt = jnp.zeros((n,), jnp.float32).at[src].add(emask_f).at[dst].add(emask_f)
    pooled = (last + nsum) / (1.0 + cnt)[:, None]
    feat = jnp.concatenate([last[node_id], pooled[node_id]], axis=1)
    return feat @ params['cls_W'] + params['cls_b']


# speedup vs baseline: 4.9844x; 3.3663x over previous
import functools

import jax
import jax.numpy as jnp
from jax import lax
from jax.experimental import pallas as pl
from jax.experimental.pallas import tpu as pltpu
from jax.experimental.pallas import tpu_sc as plsc

N = 50000
E = 800000
L = 2
D = 512
K = 64
C = 40
B = 512
T = 0.1
NUM_HOPS = 2

NACC = 50432
STRIPE = NACC // 16
TRASH = 50000
EC = 128
NCH = 196
EPAD = 2 * 16 * NCH * EC

ROWS = 2000

_MESH = plsc.VectorSubcoreMesh(core_axis_name="c", subcore_axis_name="s")
_SCPARAMS = pltpu.CompilerParams(use_tc_tiling_on_sc=False)


def _proj_body(x_ref, w1_ref, b1_ref, g_ref, bln_ref, w2_ref, b2_ref, o_ref):
    x = x_ref[...]
    h = jnp.dot(x, w1_ref[...], preferred_element_type=jnp.float32) + b1_ref[...]
    mu = jnp.mean(h, axis=-1, keepdims=True)
    var = jnp.mean((h - mu) ** 2, axis=-1, keepdims=True)
    h = (h - mu) * jax.lax.rsqrt(var + 1e-5) * g_ref[...] + bln_ref[...]
    h = jnp.maximum(h, 0.0)
    o_ref[...] = jnp.dot(h, w2_ref[...], preferred_element_type=jnp.float32) + b2_ref[...]


def _proj_pallas(x, w1, b1, g, bln, w2, b2):
    n = x.shape[0]
    return pl.pallas_call(
        _proj_body,
        grid=(n // ROWS,),
        in_specs=[
            pl.BlockSpec((ROWS, D), lambda i: (i, 0)),
            pl.BlockSpec((D, K), lambda i: (0, 0)),
            pl.BlockSpec((K,), lambda i: (0,)),
            pl.BlockSpec((K,), lambda i: (0,)),
            pl.BlockSpec((K,), lambda i: (0,)),
            pl.BlockSpec((K, K), lambda i: (0, 0)),
            pl.BlockSpec((K,), lambda i: (0,)),
        ],
        out_specs=pl.BlockSpec((ROWS, K), lambda i: (i, 0)),
        out_shape=jax.ShapeDtypeStruct((n, K), jnp.float32),
    )(x, w1, b1, g, bln, w2, b2)


def _spmm_body(W, x_hbm, gidx_hbm, ridx_hbm, z_hbm, out_hbm,
               gbuf, rbuf, rows, stage, acc, sem):
    c = lax.axis_index("c")
    s = lax.axis_index("s")

    pltpu.sync_copy(z_hbm, stage)
    pltpu.sync_copy(stage, acc.at[pl.ds(s * STRIPE, STRIPE)])
    plsc.subcore_barrier()

    @pl.loop(0, NCH)
    def _chunk(k):
        pltpu.sync_copy(gidx_hbm.at[c, s, k], gbuf)
        pltpu.sync_copy(ridx_hbm.at[c, s, k], rbuf)
        pltpu.async_copy(x_hbm.at[gbuf], rows, sem).wait()
        pltpu.sync_copy(rows, acc.at[rbuf], add=True)

    plsc.subcore_barrier()
    pltpu.sync_copy(acc.at[pl.ds(s * STRIPE, STRIPE)], stage)
    pltpu.sync_copy(stage, out_hbm.at[c, pl.ds(s * STRIPE, STRIPE)])


@functools.lru_cache(maxsize=None)
def _spmm_kern(W):
    return pl.kernel(
        functools.partial(_spmm_body, W),
        mesh=_MESH,
        compiler_params=_SCPARAMS,
        out_type=jax.ShapeDtypeStruct((2, NACC, W), jnp.float32),
        scratch_types=[
            pltpu.VMEM((EC,), jnp.int32),
            pltpu.VMEM((EC,), jnp.int32),
            pltpu.VMEM((EC, W), jnp.float32),
            pltpu.VMEM((STRIPE, W), jnp.float32),
            pltpu.VMEM_SHARED((NACC, W), jnp.float32),
            pltpu.SemaphoreType.DMA,
        ],
    )


def _spmm(x_pad, gidx, ridx):
    W = x_pad.shape[1]
    z = jnp.zeros((STRIPE, W), jnp.float32)
    out = _spmm_kern(W)(x_pad, gidx, ridx, z)
    return (out[0] + out[1])[:N]


def _pad_rows(x):
    return jnp.pad(x, ((0, NACC - N), (0, 0)))


def kernel(hidden_embeds, node_id, edge_index, params):
    src, dst = edge_index[0], edge_index[1]
    g_src = jnp.zeros((EPAD,), jnp.int32).at[:E].set(src).reshape(2, 16, NCH, EC)
    g_dst = jnp.zeros((EPAD,), jnp.int32).at[:E].set(dst).reshape(2, 16, NCH, EC)
    r_src = jnp.full((EPAD,), TRASH, jnp.int32).at[:E].set(src).reshape(2, 16, NCH, EC)
    r_dst = jnp.full((EPAD,), TRASH, jnp.int32).at[:E].set(dst).reshape(2, 16, NCH, EC)

    def mask_table(m_f):
        return jnp.broadcast_to(jnp.pad(m_f, (0, NACC - N))[:, None], (NACC, 16))

    m0 = jnp.zeros((N,), jnp.float32).at[node_id].set(1.0)
    h1 = _spmm(mask_table(m0), g_dst, r_src)[:, 0]
    m1 = ((m0 + h1) > 0).astype(jnp.float32)
    h2 = _spmm(mask_table(m1), g_dst, r_src)[:, 0]
    mask_f = ((m1 + h2) > 0).astype(jnp.float32)
    Mt = mask_table(mask_f)

    deg_dst = mask_f * _spmm(Mt, g_src, r_dst)[:, 0]
    deg_src = mask_f * _spmm(Mt, g_dst, r_src)[:, 0]
    deg = deg_dst + mask_f
    cnt = deg_dst + deg_src
    dinv = jax.lax.rsqrt(jnp.maximum(deg, 1.0))
    dm = dinv * mask_f

    def spmm64(x, gidx, ridx):
        xp = _pad_rows(x)
        return jnp.concatenate(
            [_spmm(xp[:, 16 * j:16 * (j + 1)], gidx, ridx) for j in range(4)],
            axis=1)

    def conv(x, Wp, b):
        h = x @ Wp
        s = spmm64(h * dm[:, None], g_src, r_dst)
        return dm[:, None] * s + h * (dinv * dm)[:, None] + b

    last = None
    for i in range(L):
        x = _proj_pallas(
            hidden_embeds[i],
            params[f'proj_W1_{i}'], params[f'proj_b1_{i}'],
            params[f'ln_g_{i}'], params[f'ln_b_{i}'],
            params[f'proj_W2_{i}'], params[f'proj_b2_{i}'],
        )
        if i > 0:
            a = jax.nn.sigmoid(params[f'alpha_{i}'] / T)
            x = x * a + last * (1.0 - a)
        h = jax.nn.relu(conv(x, params[f'gcn_W1_{i}'], params[f'gcn_b1_{i}']))
        last = conv(h, params[f'gcn_W2_{i}'], params[f'gcn_b2_{i}'])

    lm = last * mask_f[:, None]
    nsum = mask_f[:, None] * (spmm64(lm, g_dst, r_src) + spmm64(lm, g_src, r_dst))
    pooled = (last + nsum) / (1.0 + cnt)[:, None]
    feat = jnp.concatenate([last[node_id], pooled[node_id]], axis=1)
    return feat @ params['cls_W'] + params['cls_b']
